# baseline (speedup 1.0000x reference)
import jax
import jax.numpy as jnp
from jax import lax
from jax.experimental import pallas as pl
from jax.experimental.pallas import tpu as pltpu
from jax.experimental.pallas import tpu_sc as plsc

NP_ = 8
N_ = 500
NPAD = 512
L = 16
NC = 2
NS = 16
NW = NC * NS
IMASK = 0x1FF
KINIT = 0x7F7FFFFF


def _lrelu(t):
    return jnp.where(t >= 0, t, 0.01 * t)


def _unlace(x):
    return x.reshape(NW, NP_, L).transpose(0, 2, 1).reshape(NPAD, NP_)


NG = 2


def _sc_body(px_h, py_h, pz_h, tx_h, ty_h, tz_h,
             k1_h, k2_h,
             px_v, py_v, pz_v, tx_v, ty_v, tz_v,
             qpx, qpy, qpz, qtx, qty, qtz,
             k1b, k2b):
    wid = lax.axis_index("s")
    base = wid * (NG * L)

    pltpu.sync_copy(px_h, px_v)
    pltpu.sync_copy(py_h, py_v)
    pltpu.sync_copy(pz_h, pz_v)
    pltpu.sync_copy(tx_h, tx_v)
    pltpu.sync_copy(ty_h, ty_v)
    pltpu.sync_copy(tz_h, tz_v)
    pltpu.sync_copy(px_h.at[pl.ds(base, NG * L)], qpx)
    pltpu.sync_copy(py_h.at[pl.ds(base, NG * L)], qpy)
    pltpu.sync_copy(pz_h.at[pl.ds(base, NG * L)], qpz)
    pltpu.sync_copy(tx_h.at[pl.ds(base, NG * L)], qtx)
    pltpu.sync_copy(ty_h.at[pl.ds(base, NG * L)], qty)
    pltpu.sync_copy(tz_h.at[pl.ds(base, NG * L)], qtz)

    ap = [(qpx[pl.ds(g * L, L)], qpy[pl.ds(g * L, L)], qpz[pl.ds(g * L, L)])
          for g in range(NG)]
    at = [(qtx[pl.ds(g * L, L)], qty[pl.ds(g * L, L)], qtz[pl.ds(g * L, L)])
          for g in range(NG)]

    def insert(ks, c):
        ks = list(ks)
        for t in range(NP_):
            nk = jnp.minimum(ks[t], c)
            c = jnp.maximum(ks[t], c)
            ks[t] = nk
        return tuple(ks)

    def body(ch, carry):
        k1, k2 = carry
        k1, k2 = list(k1), list(k2)
        off = ch * L
        tvx = tx_v[pl.ds(off, L)]
        tvy = ty_v[pl.ds(off, L)]
        tvz = tz_v[pl.ds(off, L)]
        pvx = px_v[pl.ds(off, L)]
        pvy = py_v[pl.ds(off, L)]
        pvz = pz_v[pl.ds(off, L)]
        for j in range(L):
            rx, ry, rz = tvx[j], tvy[j], tvz[j]
            sx, sy, sz = pvx[j], pvy[j], pvz[j]
            m = off + j
            for g in range(NG):
                dx, dy, dz = ap[g][0] - rx, ap[g][1] - ry, ap[g][2] - rz
                d1 = dx * dx + dy * dy + dz * dz
                ex, ey, ez = at[g][0] - sx, at[g][1] - sy, at[g][2] - sz
                d2 = ex * ex + ey * ey + ez * ez
                c1 = (lax.bitcast_convert_type(d1, jnp.int32) & ~IMASK) | m
                c2 = (lax.bitcast_convert_type(d2, jnp.int32) & ~IMASK) | m
                k1[g] = insert(k1[g], c1)
                k2[g] = insert(k2[g], c2)
        return (tuple(k1), tuple(k2))

    kinit = tuple(jnp.full((L,), KINIT, jnp.int32) for _ in range(NP_))
    k1, k2 = lax.fori_loop(0, NPAD // L, body,
                           ((kinit,) * NG, (kinit,) * NG))

    for g in range(NG):
        for t in range(NP_):
            k1b[pl.ds(g * (L * NP_) + t * L, L)] = k1[g][t]
            k2b[pl.ds(g * (L * NP_) + t * L, L)] = k2[g][t]

    ob = wid * (NG * L * NP_)
    pltpu.sync_copy(k1b, k1_h.at[pl.ds(ob, NG * L * NP_)])
    pltpu.sync_copy(k2b, k2_h.at[pl.ds(ob, NG * L * NP_)])


def _sc_knn(px, py, pz, tx, ty, tz):
    i32 = jnp.int32
    grp = NG * L * NP_
    run = pl.kernel(
        _sc_body,
        out_type=(
            jax.ShapeDtypeStruct((NS * grp,), i32),
            jax.ShapeDtypeStruct((NS * grp,), i32),
        ),
        mesh=plsc.VectorSubcoreMesh(core_axis_name="c", subcore_axis_name="s",
                                    num_cores=1),
        scratch_types=(
            [pltpu.VMEM((NPAD,), jnp.float32) for _ in range(6)]
            + [pltpu.VMEM((NG * L,), jnp.float32) for _ in range(6)]
            + [pltpu.VMEM((grp,), i32), pltpu.VMEM((grp,), i32)]
        ),
    )
    return run(px, py, pz, tx, ty, tz)


def _tc_body(Pr, Gr, Cr, Tc, k1_ref, k2_ref,
             Wp1t, bp1, Wp2t, bp2, W1t, b1, W2t, b2,
             Wps1t, bps1, Wps2t, bps2,
             Wf1at, Wf1bt, bf1, Wf2at, Wf2bt, bf2,
             Wfat, Wfbt, bf, out_ref):
    dot = lambda a, b: jnp.dot(a, b, preferred_element_type=jnp.float32)
    col_iota = jax.lax.broadcasted_iota(jnp.int32, (1, NPAD), 1)
    row_iota = jax.lax.broadcasted_iota(jnp.int32, (NPAD, 1), 0)
    row_ok = (row_iota < N_).astype(jnp.float32)

    P = Pr[...]
    G = Gr[...]
    C = Cr[...]
    Tcv = Tc[...]
    k1 = k1_ref[...]
    k2 = k2_ref[...]

    cn = jnp.sum(C * C, axis=1, keepdims=True)
    tnc = jnp.sum(Tcv * Tcv, axis=0, keepdims=True)
    e1sq = cn + tnc - 2.0 * dot(C, Tcv)

    i1 = k1 & IMASK
    i2 = k2 & IMASK
    d2v = lax.bitcast_convert_type(k2 & ~IMASK, jnp.float32)
    w2 = jnp.exp(-jnp.sqrt(jnp.maximum(d2v, 0.0))) * row_ok

    A1 = jnp.zeros((NPAD, NPAD), jnp.float32)
    A2 = jnp.zeros((NPAD, NPAD), jnp.float32)
    s1 = jnp.zeros((), jnp.float32)
    for t in range(NP_):
        m1 = (col_iota == i1[:, t:t + 1]).astype(jnp.float32)
        e1d = jnp.sum(m1 * e1sq, axis=1, keepdims=True)
        w1t = jnp.exp(-jnp.sqrt(jnp.maximum(e1d, 0.0))) * row_ok
        s1 = s1 + jnp.sum(w1t)
        A1 = A1 + w1t * m1
        m2 = (col_iota == i2[:, t:t + 1]).astype(jnp.float32)
        A2 = A2 + w2[:, t:t + 1] * m2
    r1 = 1.0 / (NP_ * s1)
    r2 = 1.0 / (NP_ * jnp.sum(w2))

    def mlp2(X, Wat, ba, Wbt, bb):
        return dot(_lrelu(dot(X, Wat[...]) + ba[...]), Wbt[...]) + bb[...]

    cf = mlp2(P, Wp1t, bp1, Wp2t, bp2)
    sfull = mlp2(G, W1t, b1, W2t, b2)
    spfull = mlp2(cf, Wps1t, bps1, Wps2t, bps2)

    sf = dot(A1, sfull) * r1
    sfp = dot(A2, spfull) * r2

    final1 = dot(sf, Wf1at[...]) + dot(cf, Wf1bt[...]) + bf1[...]
    final2 = dot(sfp, Wf2at[...]) + dot(G, Wf2bt[...]) + bf2[...]
    out_ref[...] = (dot(_lrelu(final2), Wfat[...])
                    + dot(_lrelu(final1), Wfbt[...]) + bf[...])


def kernel(img_feat, cloud, cloud_tar, W1, b1, W2, b2, Wps1, bps1, Wps2, bps2,
           Wp1, bp1, Wp2, bp2, Wf1, bf1, Wf2, bf2, Wf, bf):
    f32 = jnp.float32

    def pad1(v):
        return jnp.pad(v, (0, NPAD - N_), constant_values=1e17).astype(f32)

    def padr(x):
        return jnp.pad(x, ((0, NPAD - x.shape[0]), (0, 0))).astype(f32)

    P3 = cloud[0]
    T3 = cloud_tar[0]
    C2 = cloud.reshape(3, N_)

    k1x, k2x = _sc_knn(
        pad1(P3[:, 0]), pad1(P3[:, 1]), pad1(P3[:, 2]),
        pad1(T3[:, 0]), pad1(T3[:, 1]), pad1(T3[:, 2]))

    Pr = padr(jnp.pad(P3, ((0, 0), (0, 5))))
    Tr = padr(jnp.pad(T3, ((0, 0), (0, 5))))
    Cr = padr(jnp.pad(C2.T, ((0, 0), (0, 5))))
    Tc = Tr.T[:8]
    Gr = padr(img_feat[0].T)
    row2 = lambda b: b[None, :].astype(f32)

    tc_args = (
        Pr, Gr, Cr, Tc, _unlace(k1x), _unlace(k2x),
        jnp.pad(Wp1.T, ((0, 5), (0, 0))).astype(f32), row2(bp1),
        Wp2.T.astype(f32), row2(bp2),
        W1.T.astype(f32), row2(b1), W2.T.astype(f32), row2(b2),
        Wps1.T.astype(f32), row2(bps1), Wps2.T.astype(f32), row2(bps2),
        Wf1[:, :128].T.astype(f32), Wf1[:, 128:].T.astype(f32), row2(bf1),
        Wf2[:, :128].T.astype(f32), Wf2[:, 128:].T.astype(f32), row2(bf2),
        Wf[:, :64].T.astype(f32), Wf[:, 64:].T.astype(f32), row2(bf),
    )
    out = pl.pallas_call(
        _tc_body,
        out_shape=jax.ShapeDtypeStruct((NPAD, 64), f32),
        in_specs=[pl.BlockSpec(memory_space=pltpu.VMEM) for _ in tc_args],
        out_specs=pl.BlockSpec(memory_space=pltpu.VMEM),
    )(*tc_args)

    return out[:N_].T[None]


# speedup vs baseline: 1.1608x; 1.1608x over previous
import jax
import jax.numpy as jnp
from jax import lax
from jax.experimental import pallas as pl
from jax.experimental.pallas import tpu as pltpu
from jax.experimental.pallas import tpu_sc as plsc

NP_ = 8
N_ = 500
NPAD = 512
BIG = 1e30
L = 16
NC = 2
NS = 16
NW = NC * NS
NSUB = 4
SUBN = NPAD // NSUB
IMASK = 0x1FF
KINIT = 0x7F7FFFFF


def _lrelu(t):
    return jnp.where(t >= 0, t, 0.01 * t)


def _unlace(x):
    return x.reshape(NW, NP_, L).transpose(0, 2, 1).reshape(NPAD, NP_)


def _insert(ks, c):
    ks = list(ks)
    for t in range(NP_):
        nk = jnp.minimum(ks[t], c)
        c = jnp.maximum(ks[t], c)
        ks[t] = nk
    return tuple(ks)


def _sc_body(px_h, py_h, pz_h, tx_h, ty_h, tz_h, k2_h,
             px_v, py_v, pz_v, tx_v, ty_v, tz_v,
             qtx, qty, qtz, k2b):
    wid = lax.axis_index("s") * NC + lax.axis_index("c")
    base = wid * L

    pltpu.sync_copy(px_h, px_v)
    pltpu.sync_copy(py_h, py_v)
    pltpu.sync_copy(pz_h, pz_v)
    pltpu.sync_copy(tx_h.at[pl.ds(base, L)], qtx)
    pltpu.sync_copy(ty_h.at[pl.ds(base, L)], qty)
    pltpu.sync_copy(tz_h.at[pl.ds(base, L)], qtz)

    atx, aty, atz = qtx[...], qty[...], qtz[...]

    def body(ch, carry):
        chains = list(carry)
        off = ch * L
        refs = []
        for q in range(NSUB):
            o = q * SUBN + off
            refs.append((px_v[pl.ds(o, L)], py_v[pl.ds(o, L)],
                         pz_v[pl.ds(o, L)]))
        for j in range(L):
            for q in range(NSUB):
                sx, sy, sz = refs[q][0][j], refs[q][1][j], refs[q][2][j]
                ex, ey, ez = atx - sx, aty - sy, atz - sz
                d2 = ex * ex + ey * ey + ez * ez
                m = q * SUBN + off + j
                c2 = (lax.bitcast_convert_type(d2, jnp.int32) & ~IMASK) | m
                chains[q] = _insert(chains[q], c2)
        return tuple(chains)

    kinit = tuple(jnp.full((L,), KINIT, jnp.int32) for _ in range(NP_))
    chains = lax.fori_loop(0, SUBN // L, body, (kinit,) * NSUB)

    k2 = chains[0]
    for q in range(1, NSUB):
        for t in range(NP_):
            k2 = _insert(k2, chains[q][t])

    for t in range(NP_):
        k2b[pl.ds(t * L, L)] = k2[t]
    pltpu.sync_copy(k2b, k2_h.at[pl.ds(wid * (L * NP_), L * NP_)])


def _sc_knn(px, py, pz, tx, ty, tz):
    i32 = jnp.int32
    grp = L * NP_
    run = pl.kernel(
        _sc_body,
        out_type=jax.ShapeDtypeStruct((NW * grp,), i32),
        mesh=plsc.VectorSubcoreMesh(core_axis_name="c", subcore_axis_name="s"),
        scratch_types=(
            [pltpu.VMEM((NPAD,), jnp.float32) for _ in range(6)]
            + [pltpu.VMEM((L,), jnp.float32) for _ in range(3)]
            + [pltpu.VMEM((grp,), i32)]
        ),
    )
    return run(px, py, pz, tx, ty, tz)


def _tc1_body(Pr, Gr, Cr, Tc,
              Wp1t, bp1, Wp2t, bp2, W1t, b1, W2t, b2,
              Wps1t, bps1, Wps2t, bps2,
              Wf1at, Wf1bt, bf1, Wf2bt, bf2,
              final1_ref, f2b_ref, spfull_ref):
    dot = lambda a, b: jnp.dot(a, b, preferred_element_type=jnp.float32)
    col_iota = jax.lax.broadcasted_iota(jnp.int32, (1, NPAD), 1)
    row_iota = jax.lax.broadcasted_iota(jnp.int32, (NPAD, 1), 0)

    P = Pr[...]
    G = Gr[...]
    C = Cr[...]
    Tcv = Tc[...]

    pn = jnp.sum(P * P, axis=1, keepdims=True)
    cn = jnp.sum(C * C, axis=1, keepdims=True)
    tnc = jnp.sum(Tcv * Tcv, axis=0, keepdims=True)
    d1 = pn + tnc - 2.0 * dot(P, Tcv)
    e1sq = cn + tnc - 2.0 * dot(C, Tcv)

    dd = jnp.where(col_iota >= N_, BIG, d1)
    idxs, vals = [], []
    for _ in range(NP_):
        rowmin = jnp.min(dd, axis=1, keepdims=True)
        cand = jnp.where(dd == rowmin, col_iota, NPAD)
        mstar = jnp.min(cand, axis=1, keepdims=True)
        mask = col_iota == mstar
        idxs.append(mstar)
        vals.append(jnp.sum(jnp.where(mask, e1sq, 0.0), axis=1, keepdims=True))
        dd = jnp.where(mask, BIG, dd)

    l1 = jnp.concatenate([-jnp.sqrt(jnp.maximum(v, 0.0)) for v in vals], axis=1)
    l1 = jnp.where(row_iota < N_, l1, -BIG)
    w1 = jnp.exp(l1 - jnp.max(l1))
    r1 = 1.0 / (NP_ * jnp.sum(w1))

    A1 = jnp.zeros((NPAD, NPAD), jnp.float32)
    for t in range(NP_):
        A1 = A1 + w1[:, t:t + 1] * (col_iota == idxs[t]).astype(jnp.float32)

    def mlp2(X, Wat, ba, Wbt, bb):
        return dot(_lrelu(dot(X, Wat[...]) + ba[...]), Wbt[...]) + bb[...]

    cf = mlp2(P, Wp1t, bp1, Wp2t, bp2)
    sfull = mlp2(G, W1t, b1, W2t, b2)
    spfull_ref[...] = mlp2(cf, Wps1t, bps1, Wps2t, bps2)

    sf = dot(A1, sfull) * r1
    final1_ref[...] = dot(sf, Wf1at[...]) + dot(cf, Wf1bt[...]) + bf1[...]
    f2b_ref[...] = dot(G, Wf2bt[...]) + bf2[...]


def _tc2_body(k2_ref, spfull_ref, final1_ref, f2b_ref,
              Wf2at, Wfat, Wfbt, bf, out_ref):
    dot = lambda a, b: jnp.dot(a, b, preferred_element_type=jnp.float32)
    col_iota = jax.lax.broadcasted_iota(jnp.int32, (1, NPAD), 1)
    row_iota = jax.lax.broadcasted_iota(jnp.int32, (NPAD, 1), 0)
    row_ok = (row_iota < N_).astype(jnp.float32)

    k2 = k2_ref[...]
    i2 = k2 & IMASK
    d2v = lax.bitcast_convert_type(k2 & ~IMASK, jnp.float32)
    w2 = jnp.exp(-jnp.sqrt(jnp.maximum(d2v, 0.0))) * row_ok
    r2 = 1.0 / (NP_ * jnp.sum(w2))

    A2 = jnp.zeros((NPAD, NPAD), jnp.float32)
    for t in range(NP_):
        A2 = A2 + w2[:, t:t + 1] * (col_iota == i2[:, t:t + 1]).astype(jnp.float32)

    sfp = dot(A2, spfull_ref[...]) * r2
    final2 = dot(sfp, Wf2at[...]) + f2b_ref[...]
    out_ref[...] = (dot(_lrelu(final2), Wfat[...])
                    + dot(_lrelu(final1_ref[...]), Wfbt[...]) + bf[...])


def kernel(img_feat, cloud, cloud_tar, W1, b1, W2, b2, Wps1, bps1, Wps2, bps2,
           Wp1, bp1, Wp2, bp2, Wf1, bf1, Wf2, bf2, Wf, bf):
    f32 = jnp.float32

    def pad1(v):
        return jnp.pad(v, (0, NPAD - N_), constant_values=1e17).astype(f32)

    def padr(x):
        return jnp.pad(x, ((0, NPAD - x.shape[0]), (0, 0))).astype(f32)

    P3 = cloud[0]
    T3 = cloud_tar[0]
    C2 = cloud.reshape(3, N_)

    k2x = _sc_knn(
        pad1(P3[:, 0]), pad1(P3[:, 1]), pad1(P3[:, 2]),
        pad1(T3[:, 0]), pad1(T3[:, 1]), pad1(T3[:, 2]))

    Pr = padr(jnp.pad(P3, ((0, 0), (0, 5))))
    Tr = padr(jnp.pad(T3, ((0, 0), (0, 5))))
    Cr = padr(jnp.pad(C2.T, ((0, 0), (0, 5))))
    Tc = Tr.T[:8]
    Gr = padr(img_feat[0].T)
    row2 = lambda b: b[None, :].astype(f32)

    tc1_args = (
        Pr, Gr, Cr, Tc,
        jnp.pad(Wp1.T, ((0, 5), (0, 0))).astype(f32), row2(bp1),
        Wp2.T.astype(f32), row2(bp2),
        W1.T.astype(f32), row2(b1), W2.T.astype(f32), row2(b2),
        Wps1.T.astype(f32), row2(bps1), Wps2.T.astype(f32), row2(bps2),
        Wf1[:, :128].T.astype(f32), Wf1[:, 128:].T.astype(f32), row2(bf1),
        Wf2[:, 128:].T.astype(f32), row2(bf2),
    )
    final1, f2b, spfull = pl.pallas_call(
        _tc1_body,
        out_shape=(jax.ShapeDtypeStruct((NPAD, 64), f32),
                   jax.ShapeDtypeStruct((NPAD, 64), f32),
                   jax.ShapeDtypeStruct((NPAD, 128), f32)),
        in_specs=[pl.BlockSpec(memory_space=pltpu.VMEM) for _ in tc1_args],
        out_specs=(pl.BlockSpec(memory_space=pltpu.VMEM),) * 3,
    )(*tc1_args)

    tc2_args = (
        _unlace(k2x), spfull, final1, f2b,
        Wf2[:, :128].T.astype(f32),
        Wf[:, :64].T.astype(f32), Wf[:, 64:].T.astype(f32), row2(bf),
    )
    out = pl.pallas_call(
        _tc2_body,
        out_shape=jax.ShapeDtypeStruct((NPAD, 64), f32),
        in_specs=[pl.BlockSpec(memory_space=pltpu.VMEM) for _ in tc2_args],
        out_specs=pl.BlockSpec(memory_space=pltpu.VMEM),
    )(*tc2_args)

    return out[:N_].T[None]


# speedup vs baseline: 1.4264x; 1.2288x over previous
import jax
import jax.numpy as jnp
from jax import lax
from jax.experimental import pallas as pl
from jax.experimental.pallas import tpu as pltpu
from jax.experimental.pallas import tpu_sc as plsc

NP_ = 8
N_ = 500
NPAD = 512
BIG = 1e30
L = 16
NC = 2
NS = 16
NW = NC * NS
NSUB = 4
SUBN = NPAD // NSUB
IMASK = 0x1FF
KINIT = 0x7F7FFFFF


def _lrelu(t):
    return jnp.where(t >= 0, t, 0.01 * t)


def _dg(lhs, rhs, lc, rc):
    return lax.dot_general(lhs, rhs, (((lc,), (rc,)), ((), ())),
                           preferred_element_type=jnp.float32)


def _insert(ks, c):
    ks = list(ks)
    for t in range(NP_):
        nk = jnp.minimum(ks[t], c)
        c = jnp.maximum(ks[t], c)
        ks[t] = nk
    return tuple(ks)


def _sc_body(pt_h, tt_h, k2_h,
             px_v, py_v, pz_v, qtx, qty, qtz, k2b):
    wid = lax.axis_index("s") * NC + lax.axis_index("c")
    base = wid * L

    pltpu.sync_copy(pt_h.at[pl.ds(0, NPAD)], px_v)
    pltpu.sync_copy(pt_h.at[pl.ds(NPAD, NPAD)], py_v)
    pltpu.sync_copy(pt_h.at[pl.ds(2 * NPAD, NPAD)], pz_v)
    pltpu.sync_copy(tt_h.at[pl.ds(base, L)], qtx)
    pltpu.sync_copy(tt_h.at[pl.ds(NPAD + base, L)], qty)
    pltpu.sync_copy(tt_h.at[pl.ds(2 * NPAD + base, L)], qtz)

    atx, aty, atz = qtx[...], qty[...], qtz[...]

    def body(ch, carry):
        chains = list(carry)
        off = ch * L
        refs = []
        for q in range(NSUB):
            o = q * SUBN + off
            refs.append((px_v[pl.ds(o, L)], py_v[pl.ds(o, L)],
                         pz_v[pl.ds(o, L)]))
        for j in range(L):
            for q in range(NSUB):
                sx, sy, sz = refs[q][0][j], refs[q][1][j], refs[q][2][j]
                ex, ey, ez = atx - sx, aty - sy, atz - sz
                d2 = ex * ex + ey * ey + ez * ez
                m = q * SUBN + off + j
                c2 = (lax.bitcast_convert_type(d2, jnp.int32) & ~IMASK) | m
                chains[q] = _insert(chains[q], c2)
        return tuple(chains)

    kinit = tuple(jnp.full((L,), KINIT, jnp.int32) for _ in range(NP_))
    chains = lax.fori_loop(0, SUBN // L, body, (kinit,) * NSUB)

    k2 = chains[0]
    for q in range(1, NSUB):
        for t in range(NP_):
            k2 = _insert(k2, chains[q][t])

    for t in range(NP_):
        k2b[pl.ds(t * L, L)] = k2[t]
    for t in range(NP_):
        pltpu.sync_copy(k2b.at[pl.ds(t * L, L)],
                        k2_h.at[pl.ds(t * NPAD + base, L)])


def _sc_knn(ptf, ttf):
    i32 = jnp.int32
    run = pl.kernel(
        _sc_body,
        out_type=jax.ShapeDtypeStruct((NP_ * NPAD,), i32),
        mesh=plsc.VectorSubcoreMesh(core_axis_name="c", subcore_axis_name="s"),
        scratch_types=(
            [pltpu.VMEM((NPAD,), jnp.float32) for _ in range(3)]
            + [pltpu.VMEM((L,), jnp.float32) for _ in range(3)]
            + [pltpu.VMEM((L * NP_,), i32)]
        ),
    )
    return run(ptf, ttf)


def _tc1_body(cloud_ref, tar_ref, C2_ref, img_ref,
              Wp1_r, bp1_r, Wp2_r, bp2_r, W1_r, b1_r, W2_r, b2_r,
              Wps1_r, bps1_r, Wps2_r, bps2_r,
              Wf1_r, bf1_r, Wf2_r, bf2_r,
              final1_ref, f2b_ref, spfull_ref):
    col_iota = jax.lax.broadcasted_iota(jnp.int32, (1, NPAD), 1)
    row_iota = jax.lax.broadcasted_iota(jnp.int32, (NPAD, 1), 0)
    ones8 = jnp.ones((1, 8), jnp.float32)

    P = jnp.pad(cloud_ref[...][0], ((0, NPAD - N_), (0, 5)))
    T = jnp.pad(tar_ref[...][0], ((0, NPAD - N_), (0, 5)))
    Ccm = jnp.pad(C2_ref[...], ((0, 5), (0, NPAD - N_)))
    Gcm = jnp.pad(img_ref[...][0], ((0, 0), (0, NPAD - N_)))

    pn = _dg(P * P, ones8, 1, 1)
    tn = _dg(ones8, T * T, 1, 1)
    cn = _dg(Ccm * Ccm, ones8, 0, 1)

    d1 = pn + tn - 2.0 * _dg(P, T, 1, 1)
    e1sq = cn + tn - 2.0 * _dg(Ccm, T, 0, 1)

    dd = jnp.where(col_iota >= N_, BIG, d1)
    idxs, vals = [], []
    for _ in range(NP_):
        rowmin = jnp.min(dd, axis=1, keepdims=True)
        cand = jnp.where(dd == rowmin, col_iota, NPAD)
        mstar = jnp.min(cand, axis=1, keepdims=True)
        mask = col_iota == mstar
        idxs.append(mstar)
        vals.append(jnp.sum(jnp.where(mask, e1sq, 0.0), axis=1, keepdims=True))
        dd = jnp.where(mask, BIG, dd)

    l1 = jnp.concatenate([-jnp.sqrt(jnp.maximum(v, 0.0)) for v in vals], axis=1)
    l1 = jnp.where(row_iota < N_, l1, -BIG)
    w1 = jnp.exp(l1 - jnp.max(l1))
    r1 = 1.0 / (NP_ * jnp.sum(w1))

    A1 = jnp.zeros((NPAD, NPAD), jnp.float32)
    for t in range(NP_):
        A1 = A1 + w1[:, t:t + 1] * (col_iota == idxs[t]).astype(jnp.float32)

    def mlp2(X, Wa, ba, Wb, bb, xc):
        h = _lrelu(_dg(X, Wa[...], xc, 1) + ba[...][None, :])
        return _dg(h, Wb[...], 1, 1) + bb[...][None, :]

    Wp1p = jnp.pad(Wp1_r[...], ((0, 0), (0, 5)))
    cf = mlp2(P, Wp1p, bp1_r, Wp2_r, bp2_r, 1)
    sfull = mlp2(Gcm, W1_r, b1_r, W2_r, b2_r, 0)
    spfull_ref[...] = mlp2(cf, Wps1_r, bps1_r, Wps2_r, bps2_r, 1)

    sf = _dg(A1, sfull, 1, 0) * r1
    Wf1 = Wf1_r[...]
    final1_ref[...] = (_dg(sf, Wf1[:, :128], 1, 1) + _dg(cf, Wf1[:, 128:], 1, 1)
                       + bf1_r[...][None, :])
    f2b_ref[...] = _dg(Gcm, Wf2_r[...][:, 128:], 0, 1) + bf2_r[...][None, :]


def _tc2_body(k2_ref, spfull_ref, final1_ref, f2b_ref,
              Wf2_r, Wf_r, bf_r, out_ref):
    col_iota = jax.lax.broadcasted_iota(jnp.int32, (1, NPAD), 1)
    row_iota = jax.lax.broadcasted_iota(jnp.int32, (NPAD, 1), 0)
    col_ok = (col_iota < N_).astype(jnp.float32)

    k2 = k2_ref[...]
    i2 = k2 & IMASK
    d2v = lax.bitcast_convert_type(k2 & ~IMASK, jnp.float32)
    w2 = jnp.exp(-jnp.sqrt(jnp.maximum(d2v, 0.0))) * col_ok
    r2 = 1.0 / (NP_ * jnp.sum(w2))

    B2 = jnp.zeros((NPAD, NPAD), jnp.float32)
    for t in range(NP_):
        B2 = B2 + w2[t:t + 1, :] * (row_iota == i2[t:t + 1, :]).astype(jnp.float32)

    sfp = _dg(B2, spfull_ref[...], 0, 0) * r2
    final2 = _dg(sfp, Wf2_r[...][:, :128], 1, 1) + f2b_ref[...]
    Wf = Wf_r[...]
    out = (_dg(Wf[:, :64], _lrelu(final2), 1, 1)
           + _dg(Wf[:, 64:], _lrelu(final1_ref[...]), 1, 1)
           + bf_r[...][:, None])
    out_ref[...] = out


def kernel(img_feat, cloud, cloud_tar, W1, b1, W2, b2, Wps1, bps1, Wps2, bps2,
           Wp1, bp1, Wp2, bp2, Wf1, bf1, Wf2, bf2, Wf, bf):
    f32 = jnp.float32

    def planes(pts):
        return jnp.pad(pts.T, ((0, 0), (0, NPAD - N_)),
                       constant_values=1e17).astype(f32).reshape(-1)

    P3 = cloud[0]
    T3 = cloud_tar[0]
    k2x = _sc_knn(planes(P3), planes(T3))

    C2 = cloud.reshape(3, N_)

    tc1_args = (cloud, cloud_tar, C2, img_feat,
                Wp1, bp1, Wp2, bp2, W1, b1, W2, b2,
                Wps1, bps1, Wps2, bps2, Wf1, bf1, Wf2, bf2)
    final1, f2b, spfull = pl.pallas_call(
        _tc1_body,
        out_shape=(jax.ShapeDtypeStruct((NPAD, 64), f32),
                   jax.ShapeDtypeStruct((NPAD, 64), f32),
                   jax.ShapeDtypeStruct((NPAD, 128), f32)),
        in_specs=[pl.BlockSpec(memory_space=pltpu.VMEM) for _ in tc1_args],
        out_specs=(pl.BlockSpec(memory_space=pltpu.VMEM),) * 3,
    )(*tc1_args)

    tc2_args = (k2x.reshape(NP_, NPAD), spfull, final1, f2b, Wf2, Wf, bf)
    out = pl.pallas_call(
        _tc2_body,
        out_shape=jax.ShapeDtypeStruct((64, NPAD), f32),
        in_specs=[pl.BlockSpec(memory_space=pltpu.VMEM) for _ in tc2_args],
        out_specs=pl.BlockSpec(memory_space=pltpu.VMEM),
    )(*tc2_args)

    return out[:, :N_][None]


# speedup vs baseline: 1.5618x; 1.0949x over previous
import jax
import jax.numpy as jnp
from jax import lax
from jax.experimental import pallas as pl
from jax.experimental.pallas import tpu as pltpu
from jax.experimental.pallas import tpu_sc as plsc

NP_ = 8
N_ = 500
NPAD = 512
BIG = 1e30
L = 16
NC = 2
NS = 16
NW = NC * NS
NSUB = 4
SUBN = NPAD // NSUB
IMASK = 0x1FF
KINIT = 0x7F7FFFFF


def _lrelu(t):
    return jnp.where(t >= 0, t, 0.01 * t)


def _dg(lhs, rhs, lc, rc):
    return lax.dot_general(lhs, rhs, (((lc,), (rc,)), ((), ())),
                           preferred_element_type=jnp.float32)


def _insert(ks, c):
    ks = list(ks)
    for t in range(NP_):
        nk = jnp.minimum(ks[t], c)
        c = jnp.maximum(ks[t], c)
        ks[t] = nk
    return tuple(ks)


def _sc_body(pt_h, tt_h, k2_h,
             px_v, py_v, pz_v, qtx, qty, qtz, k2b):
    wid = lax.axis_index("s") * NC + lax.axis_index("c")
    base = wid * L

    pltpu.sync_copy(pt_h.at[pl.ds(0, NPAD)], px_v.at[pl.ds(0, NPAD)])
    pltpu.sync_copy(pt_h.at[pl.ds(NPAD, NPAD)], py_v.at[pl.ds(0, NPAD)])
    pltpu.sync_copy(pt_h.at[pl.ds(2 * NPAD, NPAD)], pz_v.at[pl.ds(0, NPAD)])
    pltpu.sync_copy(tt_h.at[pl.ds(base, L)], qtx)
    pltpu.sync_copy(tt_h.at[pl.ds(NPAD + base, L)], qty)
    pltpu.sync_copy(tt_h.at[pl.ds(2 * NPAD + base, L)], qtz)

    atx, aty, atz = qtx[...], qty[...], qtz[...]

    JB = 8

    def body(ch, carry):
        chains = list(carry)
        off = ch * JB
        refs = []
        for q in range(NSUB):
            o = q * SUBN + off
            refs.append((px_v[pl.ds(o, L)], py_v[pl.ds(o, L)],
                         pz_v[pl.ds(o, L)]))
        for j in range(JB):
            for q in range(NSUB):
                sx, sy, sz = refs[q][0][j], refs[q][1][j], refs[q][2][j]
                ex, ey, ez = atx - sx, aty - sy, atz - sz
                d2 = ex * ex + ey * ey + ez * ez
                m = q * SUBN + off + j
                c2 = (lax.bitcast_convert_type(d2, jnp.int32) & ~IMASK) | m
                chains[q] = _insert(chains[q], c2)
        return tuple(chains)

    kinit = tuple(jnp.full((L,), KINIT, jnp.int32) for _ in range(NP_))
    chains = lax.fori_loop(0, SUBN // JB, body, (kinit,) * NSUB)

    k2 = chains[0]
    for q in range(1, NSUB):
        for t in range(NP_):
            k2 = _insert(k2, chains[q][t])

    for t in range(NP_):
        k2b[pl.ds(t * L, L)] = k2[t]
    for t in range(NP_):
        pltpu.sync_copy(k2b.at[pl.ds(t * L, L)],
                        k2_h.at[t, pl.ds(base, L)])


def _sc_knn(ptf, ttf):
    i32 = jnp.int32
    run = pl.kernel(
        _sc_body,
        out_type=jax.ShapeDtypeStruct((NP_, NPAD), i32),
        mesh=plsc.VectorSubcoreMesh(core_axis_name="c", subcore_axis_name="s"),
        scratch_types=(
            [pltpu.VMEM((NPAD + L,), jnp.float32) for _ in range(3)]
            + [pltpu.VMEM((L,), jnp.float32) for _ in range(3)]
            + [pltpu.VMEM((L * NP_,), i32)]
        ),
    )
    return run(ptf, ttf)


def _tc1_body(cloud_ref, tar_ref, C2_ref, img_ref,
              Wp1_r, bp1_r, Wp2_r, bp2_r, W1_r, b1_r, W2_r, b2_r,
              Wps1_r, bps1_r, Wps2_r, bps2_r,
              Wf1_r, bf1_r, Wf2_r, bf2_r,
              final1_ref, f2b_ref, spfull_ref):
    col_iota = jax.lax.broadcasted_iota(jnp.int32, (1, NPAD), 1)
    row_iota = jax.lax.broadcasted_iota(jnp.int32, (NPAD, 1), 0)
    ones8 = jnp.ones((1, 8), jnp.float32)

    P = jnp.pad(cloud_ref[...][0], ((0, NPAD - N_), (0, 5)))
    T = jnp.pad(tar_ref[...][0], ((0, NPAD - N_), (0, 5)))
    Ccm = jnp.pad(C2_ref[...], ((0, 5), (0, NPAD - N_)))
    Gcm = jnp.pad(img_ref[...][0], ((0, 0), (0, NPAD - N_)))

    pn = _dg(P * P, ones8, 1, 1)
    tn = _dg(ones8, T * T, 1, 1)
    cn = _dg(Ccm * Ccm, ones8, 0, 1)

    d1 = pn + tn - 2.0 * _dg(P, T, 1, 1)
    e1sq = cn + tn - 2.0 * _dg(Ccm, T, 0, 1)

    dd = jnp.where(col_iota >= N_, BIG, d1)
    idxs, vals = [], []
    for _ in range(NP_):
        rowmin = jnp.min(dd, axis=1, keepdims=True)
        cand = jnp.where(dd == rowmin, col_iota, NPAD)
        mstar = jnp.min(cand, axis=1, keepdims=True)
        mask = col_iota == mstar
        idxs.append(mstar)
        vals.append(jnp.sum(jnp.where(mask, e1sq, 0.0), axis=1, keepdims=True))
        dd = jnp.where(mask, BIG, dd)

    l1 = jnp.concatenate([-jnp.sqrt(jnp.maximum(v, 0.0)) for v in vals], axis=1)
    l1 = jnp.where(row_iota < N_, l1, -BIG)
    w1 = jnp.exp(l1 - jnp.max(l1))
    r1 = 1.0 / (NP_ * jnp.sum(w1))

    A1 = jnp.zeros((NPAD, NPAD), jnp.float32)
    for t in range(NP_):
        A1 = A1 + w1[:, t:t + 1] * (col_iota == idxs[t]).astype(jnp.float32)

    def mlp2(X, Wa, ba, Wb, bb, xc):
        h = _lrelu(_dg(X, Wa[...], xc, 1) + ba[...][None, :])
        return _dg(h, Wb[...], 1, 1) + bb[...][None, :]

    Wp1p = jnp.pad(Wp1_r[...], ((0, 0), (0, 5)))
    cf = mlp2(P, Wp1p, bp1_r, Wp2_r, bp2_r, 1)
    sfull = mlp2(Gcm, W1_r, b1_r, W2_r, b2_r, 0)
    spfull_ref[...] = mlp2(cf, Wps1_r, bps1_r, Wps2_r, bps2_r, 1)

    sf = _dg(A1, sfull, 1, 0) * r1
    Wf1 = Wf1_r[...]
    final1_ref[...] = (_dg(sf, Wf1[:, :128], 1, 1) + _dg(cf, Wf1[:, 128:], 1, 1)
                       + bf1_r[...][None, :])
    f2b_ref[...] = _dg(Gcm, Wf2_r[...][:, 128:], 0, 1) + bf2_r[...][None, :]


def _tc2_body(k2_ref, spfull_ref, final1_ref, f2b_ref,
              Wf2_r, Wf_r, bf_r, out_ref):
    col_iota = jax.lax.broadcasted_iota(jnp.int32, (1, NPAD), 1)
    row_iota = jax.lax.broadcasted_iota(jnp.int32, (NPAD, 1), 0)
    col_ok = (col_iota < N_).astype(jnp.float32)

    k2 = k2_ref[...]
    i2 = k2 & IMASK
    d2v = lax.bitcast_convert_type(k2 & ~IMASK, jnp.float32)
    w2 = jnp.exp(-jnp.sqrt(jnp.maximum(d2v, 0.0))) * col_ok
    r2 = 1.0 / (NP_ * jnp.sum(w2))

    B2 = jnp.zeros((NPAD, NPAD), jnp.float32)
    for t in range(NP_):
        B2 = B2 + w2[t:t + 1, :] * (row_iota == i2[t:t + 1, :]).astype(jnp.float32)

    sfp = _dg(B2, spfull_ref[...], 0, 0) * r2
    final2 = _dg(sfp, Wf2_r[...][:, :128], 1, 1) + f2b_ref[...]
    Wf = Wf_r[...]
    out = (_dg(Wf[:, :64], _lrelu(final2), 1, 1)
           + _dg(Wf[:, 64:], _lrelu(final1_ref[...]), 1, 1)
           + bf_r[...][:, None])
    out_ref[...] = out[:, :N_][None]


def kernel(img_feat, cloud, cloud_tar, W1, b1, W2, b2, Wps1, bps1, Wps2, bps2,
           Wp1, bp1, Wp2, bp2, Wf1, bf1, Wf2, bf2, Wf, bf):
    f32 = jnp.float32

    def planes(pts):
        return jnp.pad(pts.T, ((0, 0), (0, NPAD - N_)),
                       constant_values=1e17).astype(f32).reshape(-1)

    P3 = cloud[0]
    T3 = cloud_tar[0]
    k2x = _sc_knn(planes(P3), planes(T3))

    C2 = cloud.reshape(3, N_)

    tc1_args = (cloud, cloud_tar, C2, img_feat,
                Wp1, bp1, Wp2, bp2, W1, b1, W2, b2,
                Wps1, bps1, Wps2, bps2, Wf1, bf1, Wf2, bf2)
    final1, f2b, spfull = pl.pallas_call(
        _tc1_body,
        out_shape=(jax.ShapeDtypeStruct((NPAD, 64), f32),
                   jax.ShapeDtypeStruct((NPAD, 64), f32),
                   jax.ShapeDtypeStruct((NPAD, 128), f32)),
        in_specs=[pl.BlockSpec(memory_space=pltpu.VMEM) for _ in tc1_args],
        out_specs=(pl.BlockSpec(memory_space=pltpu.VMEM),) * 3,
    )(*tc1_args)

    tc2_args = (k2x, spfull, final1, f2b, Wf2, Wf, bf)
    out = pl.pallas_call(
        _tc2_body,
        out_shape=jax.ShapeDtypeStruct((1, 64, N_), f32),
        in_specs=[pl.BlockSpec(memory_space=pltpu.VMEM) for _ in tc2_args],
        out_specs=pl.BlockSpec(memory_space=pltpu.VMEM),
    )(*tc2_args)

    return out


# speedup vs baseline: 1.5673x; 1.0035x over previous
import jax
import jax.numpy as jnp
from jax import lax
from jax.experimental import pallas as pl
from jax.experimental.pallas import tpu as pltpu
from jax.experimental.pallas import tpu_sc as plsc

NP_ = 8
N_ = 500
NPAD = 512
BIG = 1e30
L = 16
NC = 2
NS = 16
NW = NC * NS
NSUB = 2
SUBN = NPAD // NSUB
IMASK = 0x1FF
KINIT = 0x7F7FFFFF


def _lrelu(t):
    return jnp.where(t >= 0, t, 0.01 * t)


def _dg(lhs, rhs, lc, rc):
    return lax.dot_general(lhs, rhs, (((lc,), (rc,)), ((), ())),
                           preferred_element_type=jnp.float32)


def _insert(ks, c):
    ks = list(ks)
    for t in range(NP_):
        nk = jnp.minimum(ks[t], c)
        c = jnp.maximum(ks[t], c)
        ks[t] = nk
    return tuple(ks)


def _sc_body(pt_h, tt_h, k2_h,
             px_v, py_v, pz_v, qtx, qty, qtz, k2b):
    wid = lax.axis_index("s") * NC + lax.axis_index("c")
    base = wid * L

    pltpu.sync_copy(pt_h.at[pl.ds(0, NPAD)], px_v.at[pl.ds(0, NPAD)])
    pltpu.sync_copy(pt_h.at[pl.ds(NPAD, NPAD)], py_v.at[pl.ds(0, NPAD)])
    pltpu.sync_copy(pt_h.at[pl.ds(2 * NPAD, NPAD)], pz_v.at[pl.ds(0, NPAD)])
    pltpu.sync_copy(tt_h.at[pl.ds(base, L)], qtx)
    pltpu.sync_copy(tt_h.at[pl.ds(NPAD + base, L)], qty)
    pltpu.sync_copy(tt_h.at[pl.ds(2 * NPAD + base, L)], qtz)

    atx, aty, atz = qtx[...], qty[...], qtz[...]

    JB = 8

    def body(ch, carry):
        chains = list(carry)
        off = ch * JB
        refs = []
        for q in range(NSUB):
            o = q * SUBN + off
            refs.append((px_v[pl.ds(o, L)], py_v[pl.ds(o, L)],
                         pz_v[pl.ds(o, L)]))
        for j in range(JB):
            for q in range(NSUB):
                sx, sy, sz = refs[q][0][j], refs[q][1][j], refs[q][2][j]
                ex, ey, ez = atx - sx, aty - sy, atz - sz
                d2 = ex * ex + ey * ey + ez * ez
                m = q * SUBN + off + j
                c2 = (lax.bitcast_convert_type(d2, jnp.int32) & ~IMASK) | m
                chains[q] = _insert(chains[q], c2)
        return tuple(chains)

    kinit = tuple(jnp.full((L,), KINIT, jnp.int32) for _ in range(NP_))
    chains = lax.fori_loop(0, SUBN // JB, body, (kinit,) * NSUB)

    k2 = chains[0]
    for q in range(1, NSUB):
        for t in range(NP_):
            k2 = _insert(k2, chains[q][t])

    for t in range(NP_):
        k2b[pl.ds(t * L, L)] = k2[t]
    for t in range(NP_):
        pltpu.sync_copy(k2b.at[pl.ds(t * L, L)],
                        k2_h.at[t, pl.ds(base, L)])


def _sc_knn(ptf, ttf):
    i32 = jnp.int32
    run = pl.kernel(
        _sc_body,
        out_type=jax.ShapeDtypeStruct((NP_, NPAD), i32),
        mesh=plsc.VectorSubcoreMesh(core_axis_name="c", subcore_axis_name="s"),
        scratch_types=(
            [pltpu.VMEM((NPAD + L,), jnp.float32) for _ in range(3)]
            + [pltpu.VMEM((L,), jnp.float32) for _ in range(3)]
            + [pltpu.VMEM((L * NP_,), i32)]
        ),
    )
    return run(ptf, ttf)


def _tc1_body(cloud_ref, tar_ref, C2_ref, img_ref,
              Wp1_r, bp1_r, Wp2_r, bp2_r, W1_r, b1_r, W2_r, b2_r,
              Wps1_r, bps1_r, Wps2_r, bps2_r,
              Wf1_r, bf1_r, Wf2_r, bf2_r,
              final1_ref, f2b_ref, spfull_ref):
    col_iota = jax.lax.broadcasted_iota(jnp.int32, (1, NPAD), 1)
    row_iota = jax.lax.broadcasted_iota(jnp.int32, (NPAD, 1), 0)
    ones8 = jnp.ones((1, 8), jnp.float32)

    P = jnp.pad(cloud_ref[...][0], ((0, NPAD - N_), (0, 5)))
    T = jnp.pad(tar_ref[...][0], ((0, NPAD - N_), (0, 5)))
    Ccm = jnp.pad(C2_ref[...], ((0, 5), (0, NPAD - N_)))
    Gcm = jnp.pad(img_ref[...][0], ((0, 0), (0, NPAD - N_)))

    pn = _dg(P * P, ones8, 1, 1)
    tn = _dg(ones8, T * T, 1, 1)
    cn = _dg(Ccm * Ccm, ones8, 0, 1)

    d1 = pn + tn - 2.0 * _dg(P, T, 1, 1)
    e1sq = cn + tn - 2.0 * _dg(Ccm, T, 0, 1)

    dd = jnp.where(col_iota >= N_, BIG, d1)
    idxs, vals = [], []
    for _ in range(NP_):
        rowmin = jnp.min(dd, axis=1, keepdims=True)
        cand = jnp.where(dd == rowmin, col_iota, NPAD)
        mstar = jnp.min(cand, axis=1, keepdims=True)
        mask = col_iota == mstar
        idxs.append(mstar)
        vals.append(jnp.sum(jnp.where(mask, e1sq, 0.0), axis=1, keepdims=True))
        dd = jnp.where(mask, BIG, dd)

    l1 = jnp.concatenate([-jnp.sqrt(jnp.maximum(v, 0.0)) for v in vals], axis=1)
    l1 = jnp.where(row_iota < N_, l1, -BIG)
    w1 = jnp.exp(l1 - jnp.max(l1))
    r1 = 1.0 / (NP_ * jnp.sum(w1))

    A1 = jnp.zeros((NPAD, NPAD), jnp.float32)
    for t in range(NP_):
        A1 = A1 + w1[:, t:t + 1] * (col_iota == idxs[t]).astype(jnp.float32)

    def mlp2(X, Wa, ba, Wb, bb, xc):
        h = _lrelu(_dg(X, Wa[...], xc, 1) + ba[...][None, :])
        return _dg(h, Wb[...], 1, 1) + bb[...][None, :]

    Wp1p = jnp.pad(Wp1_r[...], ((0, 0), (0, 5)))
    cf = mlp2(P, Wp1p, bp1_r, Wp2_r, bp2_r, 1)
    sfull = mlp2(Gcm, W1_r, b1_r, W2_r, b2_r, 0)
    spfull_ref[...] = mlp2(cf, Wps1_r, bps1_r, Wps2_r, bps2_r, 1)

    sf = _dg(A1, sfull, 1, 0) * r1
    Wf1 = Wf1_r[...]
    final1_ref[...] = (_dg(sf, Wf1[:, :128], 1, 1) + _dg(cf, Wf1[:, 128:], 1, 1)
                       + bf1_r[...][None, :])
    f2b_ref[...] = _dg(Gcm, Wf2_r[...][:, 128:], 0, 1) + bf2_r[...][None, :]


def _tc2_body(k2_ref, spfull_ref, final1_ref, f2b_ref,
              Wf2_r, Wf_r, bf_r, out_ref):
    col_iota = jax.lax.broadcasted_iota(jnp.int32, (1, NPAD), 1)
    row_iota = jax.lax.broadcasted_iota(jnp.int32, (NPAD, 1), 0)
    col_ok = (col_iota < N_).astype(jnp.float32)

    k2 = k2_ref[...]
    i2 = k2 & IMASK
    d2v = lax.bitcast_convert_type(k2 & ~IMASK, jnp.float32)
    w2 = jnp.exp(-jnp.sqrt(jnp.maximum(d2v, 0.0))) * col_ok
    r2 = 1.0 / (NP_ * jnp.sum(w2))

    B2 = jnp.zeros((NPAD, NPAD), jnp.float32)
    for t in range(NP_):
        B2 = B2 + w2[t:t + 1, :] * (row_iota == i2[t:t + 1, :]).astype(jnp.float32)

    sfp = _dg(B2, spfull_ref[...], 0, 0) * r2
    final2 = _dg(sfp, Wf2_r[...][:, :128], 1, 1) + f2b_ref[...]
    Wf = Wf_r[...]
    out = (_dg(Wf[:, :64], _lrelu(final2), 1, 1)
           + _dg(Wf[:, 64:], _lrelu(final1_ref[...]), 1, 1)
           + bf_r[...][:, None])
    out_ref[...] = out[:, :N_][None]


def kernel(img_feat, cloud, cloud_tar, W1, b1, W2, b2, Wps1, bps1, Wps2, bps2,
           Wp1, bp1, Wp2, bp2, Wf1, bf1, Wf2, bf2, Wf, bf):
    f32 = jnp.float32

    def planes(pts):
        return jnp.pad(pts.T, ((0, 0), (0, NPAD - N_)),
                       constant_values=1e17).astype(f32).reshape(-1)

    P3 = cloud[0]
    T3 = cloud_tar[0]
    k2x = _sc_knn(planes(P3), planes(T3))

    C2 = cloud.reshape(3, N_)

    tc1_args = (cloud, cloud_tar, C2, img_feat,
                Wp1, bp1, Wp2, bp2, W1, b1, W2, b2,
                Wps1, bps1, Wps2, bps2, Wf1, bf1, Wf2, bf2)
    final1, f2b, spfull = pl.pallas_call(
        _tc1_body,
        out_shape=(jax.ShapeDtypeStruct((NPAD, 64), f32),
                   jax.ShapeDtypeStruct((NPAD, 64), f32),
                   jax.ShapeDtypeStruct((NPAD, 128), f32)),
        in_specs=[pl.BlockSpec(memory_space=pltpu.VMEM) for _ in tc1_args],
        out_specs=(pl.BlockSpec(memory_space=pltpu.VMEM),) * 3,
    )(*tc1_args)

    tc2_args = (k2x, spfull, final1, f2b, Wf2, Wf, bf)
    out = pl.pallas_call(
        _tc2_body,
        out_shape=jax.ShapeDtypeStruct((1, 64, N_), f32),
        in_specs=[pl.BlockSpec(memory_space=pltpu.VMEM) for _ in tc2_args],
        out_specs=pl.BlockSpec(memory_space=pltpu.VMEM),
    )(*tc2_args)

    return out
